# baseline (speedup 1.0000x reference)
import functools

import jax
import jax.numpy as jnp
from jax import lax
from jax.experimental import pallas as pl
from jax.experimental.pallas import tpu as pltpu
from jax.experimental.pallas import tpu_sc as plsc

N_NODES = 50000
D = 256
M = 50000
F = 10000
G = 256

NC, NS, L = 2, 16, 16
NW = NC * NS
B = 32
NBUF = 7
NB = 49
CHUNK = B * NB
APAD = 1792
M_PAD = CHUNK * NW
FR = 640
F_PAD = FR * NS


def _iota():
    return lax.iota(jnp.int32, L)


def _bsearch(ref, size, v, upper):
    lo = jnp.zeros((L,), jnp.int32)
    hi = jnp.full((L,), size, jnp.int32)

    def step(_, lh):
        lo, hi = lh
        active = lo < hi
        mid = (lo + hi) // 2
        val = plsc.load_gather(ref, [jnp.minimum(mid, size - 1)])
        pred = (val <= v) if upper else (val < v)
        lo2 = jnp.where(active & pred, mid + 1, lo)
        hi2 = jnp.where(active & (~pred), mid, hi)
        return (lo2, hi2)

    lo, hi = lax.fori_loop(0, size.bit_length(), step, (lo, hi))
    return lo


def _prep_body(seg_hbm, bids_hbm, w_hbm, g_hbm, gcnt_hbm,
               seg_v, bids_v, fcnt_v, fct_v, w_v, gx_v, gcnt_v, fragsh):
    cid = lax.axis_index("c")
    sid = lax.axis_index("s")
    wid = sid * NC + cid
    base = wid * CHUNK

    pltpu.sync_copy(seg_hbm, seg_v)
    pltpu.sync_copy(bids_hbm, bids_v)

    fb = FR * sid

    def fc_body(g, _):
        f16 = jnp.minimum(fb + g * L + _iota(), F - 1)
        lb = _bsearch(seg_v, M_PAD, f16, upper=False)
        ub = _bsearch(seg_v, M_PAD, f16, upper=True)
        fcnt_v[pl.ds(g * L, L)] = (ub - lb).astype(jnp.float32)
        return 0

    lax.fori_loop(0, FR // L, fc_body, 0)
    pltpu.sync_copy(fcnt_v, fragsh.at[pl.ds(fb, FR)])

    gv16 = sid * L + _iota()
    glb = _bsearch(bids_v, F, gv16, upper=False)
    gub = _bsearch(bids_v, F, gv16, upper=True)
    gcnt_v[...] = (gub - glb).astype(jnp.float32)
    pltpu.sync_copy(gcnt_v, gcnt_hbm.at[cid, pl.ds(sid * L, L)])

    plsc.subcore_barrier()
    pltpu.sync_copy(fragsh, fct_v)

    def wg_body(k, _):
        seg16 = seg_v[pl.ds(base + k * L, L)]
        segc = jnp.minimum(seg16, F - 1)
        cnt = plsc.load_gather(fct_v, [segc])
        w = 1.0 / cnt
        pos = base + k * L + _iota()
        w_v[pl.ds(k * L, L)] = jnp.where(pos < M, w, 0.0)
        gi = plsc.load_gather(bids_v, [segc])
        gx_v[pl.ds(k * L, L)] = gi * D
        return 0

    lax.fori_loop(0, CHUNK // L, wg_body, 0)
    pltpu.sync_copy(w_v, w_hbm.at[pl.ds(base, CHUNK)])
    pltpu.sync_copy(gx_v, g_hbm.at[pl.ds(base, CHUNK)])


def _main_body(node_hbm, atom_hbm, w_hbm, g_hbm, part_hbm,
               acc_v, w_v, gx_v, atom_v, *rest):
    bufs = rest[:NBUF]
    sems = rest[NBUF:2 * NBUF]
    cid = lax.axis_index("c")
    sid = lax.axis_index("s")
    wid = sid * NC + cid
    base = wid * CHUNK

    pltpu.sync_copy(atom_hbm.at[wid], atom_v)
    pltpu.sync_copy(w_hbm.at[pl.ds(base, CHUNK)], w_v)
    pltpu.sync_copy(g_hbm.at[pl.ds(base, CHUNK)], gx_v)

    def fire(b, i):
        pltpu.async_copy(node_hbm.at[atom_v.at[b]], bufs[i], sems[i])

    def drain(i):
        pltpu.make_async_copy(node_hbm.at[atom_v.at[0]], bufs[i],
                              sems[i]).wait()

    for i in range(NBUF):
        fire(i, i)

    zv = jnp.zeros((L,), jnp.float32)

    def z_body(i, _):
        acc_v[pl.ds(i * L, L)] = zv
        return 0

    lax.fori_loop(0, (G * D) // L, z_body, 0)

    def process(b, buf):
        def row(r, _):
            e = b * B + r
            g16 = plsc.load_gather(gx_v, [jnp.broadcast_to(e, (L,))])
            w16 = plsc.load_gather(w_v, [jnp.broadcast_to(e, (L,))])
            idx0 = g16 + _iota()
            for q in range(D // 32):
                v = buf[r, pl.ds(q * L, L)]
                lo = plsc.bitcast(v << 16, jnp.float32)
                hi = plsc.bitcast(v & jnp.int32(-65536), jnp.float32)
                col0 = q * L
                plsc.addupdate_scatter(acc_v, [idx0 + col0], lo * w16)
                plsc.addupdate_scatter(acc_v, [idx0 + (col0 + 128)], hi * w16)
            return 0

        lax.fori_loop(0, B, row, 0)

    def ring(t, _):
        for i in range(NBUF):
            b = t * NBUF + i
            drain(i)
            process(b, bufs[i])
            fire(b + NBUF, i)
        return 0

    lax.fori_loop(0, NB // NBUF, ring, 0)
    for i in range(NBUF):
        drain(i)

    pltpu.sync_copy(acc_v, part_hbm.at[wid])


@functools.lru_cache(maxsize=None)
def _build_prep():
    return functools.partial(
        pl.kernel,
        out_type=[
            jax.ShapeDtypeStruct((M_PAD,), jnp.float32),
            jax.ShapeDtypeStruct((M_PAD,), jnp.int32),
            jax.ShapeDtypeStruct((NC, G), jnp.float32),
        ],
        mesh=plsc.VectorSubcoreMesh(core_axis_name="c", subcore_axis_name="s",
                                    num_cores=NC, num_subcores=NS),
        compiler_params=pltpu.CompilerParams(needs_layout_passes=False),
        scratch_types=[
            pltpu.VMEM((M_PAD,), jnp.int32),
            pltpu.VMEM((F,), jnp.int32),
            pltpu.VMEM((FR,), jnp.float32),
            pltpu.VMEM((F_PAD,), jnp.float32),
            pltpu.VMEM((CHUNK,), jnp.float32),
            pltpu.VMEM((CHUNK,), jnp.int32),
            pltpu.VMEM((L,), jnp.float32),
            pltpu.VMEM_SHARED((F_PAD,), jnp.float32),
        ],
    )(_prep_body)


@functools.lru_cache(maxsize=None)
def _build_main():
    return functools.partial(
        pl.kernel,
        out_type=jax.ShapeDtypeStruct((NW, G * D), jnp.float32),
        mesh=plsc.VectorSubcoreMesh(core_axis_name="c", subcore_axis_name="s",
                                    num_cores=NC, num_subcores=NS),
        compiler_params=pltpu.CompilerParams(needs_layout_passes=False),
        scratch_types=(
            [
                pltpu.VMEM((G * D,), jnp.float32),
                pltpu.VMEM((CHUNK,), jnp.float32),
                pltpu.VMEM((CHUNK,), jnp.int32),
                pltpu.VMEM((NB + NBUF, B), jnp.int32),
            ]
            + [pltpu.VMEM((B, D // 2), jnp.int32) for _ in range(NBUF)]
            + [pltpu.SemaphoreType.DMA for _ in range(NBUF)]
        ),
    )(_main_body)


def _cast_body(x_ref, o_ref):
    lobits = lax.bitcast_convert_type(
        x_ref[:, :128].astype(jnp.bfloat16).astype(jnp.float32), jnp.uint32)
    hibits = lax.bitcast_convert_type(
        x_ref[:, 128:].astype(jnp.bfloat16).astype(jnp.float32), jnp.uint32)
    o_ref[...] = ((lobits >> 16) | (hibits & jnp.uint32(0xFFFF0000))
                  ).astype(jnp.int32)


_CAST_ROWS = 2000
_cast = pl.pallas_call(
    _cast_body,
    grid=(N_NODES // _CAST_ROWS,),
    in_specs=[pl.BlockSpec((_CAST_ROWS, D), lambda i: (i, 0))],
    out_specs=pl.BlockSpec((_CAST_ROWS, D // 2), lambda i: (i, 0)),
    out_shape=jax.ShapeDtypeStruct((N_NODES, D // 2), jnp.int32),
)


def _combine_body(part_ref, gcnt_ref, out_ref):
    cnt = jnp.maximum(gcnt_ref[0], 1.0)
    out_ref[...] = jnp.sum(part_ref[...], axis=0) / cnt[:, None]


_combine = pl.pallas_call(
    _combine_body,
    out_shape=jax.ShapeDtypeStruct((G, D), jnp.float32),
)


def kernel(node_feature, group_atom_idxs_1d, macro_node_scatter_idxs,
           fragment_batch_ids):
    seg = jnp.pad(macro_node_scatter_idxs.astype(jnp.int32),
                  (0, M_PAD - M), constant_values=F)
    atom = jnp.pad(group_atom_idxs_1d.astype(jnp.int32),
                   (0, M_PAD - M)).reshape(NW, NB, B)
    atom = jnp.concatenate(
        [atom, jnp.zeros((NW, NBUF, B), jnp.int32)], axis=1)
    bids = fragment_batch_ids.astype(jnp.int32)

    w, gx, gcnt = _build_prep()(seg, bids)
    node_i32 = _cast(node_feature)
    part = _build_main()(node_i32, atom, w, gx)
    return _combine(part.reshape(NW, G, D), gcnt)


# speedup vs baseline: 1.7726x; 1.7726x over previous
import functools

import jax
import jax.numpy as jnp
from jax import lax
from jax.experimental import pallas as pl
from jax.experimental.pallas import tpu as pltpu
from jax.experimental.pallas import tpu_sc as plsc

N_NODES = 50000
D = 256
M = 50000
F = 10000
G = 256

NC, NS, L = 2, 16, 16
NW = NC * NS
B = 8
NBUF = 7
NB = 196
CHUNK = B * NB
APAD = 1792
M_PAD = CHUNK * NW
FR = 640
F_PAD = FR * NS


def _iota():
    return lax.iota(jnp.int32, L)


def _bsearch(ref, size, v, upper):
    lo = jnp.zeros((L,), jnp.int32)
    hi = jnp.full((L,), size, jnp.int32)

    def step(_, lh):
        lo, hi = lh
        active = lo < hi
        mid = (lo + hi) // 2
        val = plsc.load_gather(ref, [jnp.minimum(mid, size - 1)])
        pred = (val <= v) if upper else (val < v)
        lo2 = jnp.where(active & pred, mid + 1, lo)
        hi2 = jnp.where(active & (~pred), mid, hi)
        return (lo2, hi2)

    lo, hi = lax.fori_loop(0, size.bit_length(), step, (lo, hi))
    return lo


def _prep_body(seg_hbm, bids_hbm, w_hbm, g_hbm, gcnt_hbm,
               seg_v, bids_v, fcnt_v, fct_v, w_v, gx_v, gcnt_v, fragsh):
    cid = lax.axis_index("c")
    sid = lax.axis_index("s")
    wid = sid * NC + cid
    base = wid * CHUNK

    pltpu.sync_copy(seg_hbm, seg_v)
    pltpu.sync_copy(bids_hbm, bids_v)

    fb = FR * sid

    def fc_body(g, _):
        f16 = jnp.minimum(fb + g * L + _iota(), F - 1)
        lb = _bsearch(seg_v, M_PAD, f16, upper=False)
        ub = _bsearch(seg_v, M_PAD, f16, upper=True)
        fcnt_v[pl.ds(g * L, L)] = (ub - lb).astype(jnp.float32)
        return 0

    lax.fori_loop(0, FR // L, fc_body, 0)
    pltpu.sync_copy(fcnt_v, fragsh.at[pl.ds(fb, FR)])

    gv16 = sid * L + _iota()
    glb = _bsearch(bids_v, F, gv16, upper=False)
    gub = _bsearch(bids_v, F, gv16, upper=True)
    gcnt_v[...] = (gub - glb).astype(jnp.float32)
    pltpu.sync_copy(gcnt_v, gcnt_hbm.at[cid, pl.ds(sid * L, L)])

    plsc.subcore_barrier()
    pltpu.sync_copy(fragsh, fct_v)

    def wg_body(k, _):
        seg16 = seg_v[pl.ds(base + k * L, L)]
        segc = jnp.minimum(seg16, F - 1)
        cnt = plsc.load_gather(fct_v, [segc])
        w = 1.0 / cnt
        pos = base + k * L + _iota()
        w_v[pl.ds(k * L, L)] = jnp.where(pos < M, w, 0.0)
        gi = plsc.load_gather(bids_v, [segc])
        gx_v[pl.ds(k * L, L)] = gi * D
        return 0

    lax.fori_loop(0, CHUNK // L, wg_body, 0)
    pltpu.sync_copy(w_v, w_hbm.at[pl.ds(base, CHUNK)])
    pltpu.sync_copy(gx_v, g_hbm.at[pl.ds(base, CHUNK)])


def _main_body(node_hbm, atom_hbm, w_hbm, g_hbm, part_hbm,
               acc_v, w_v, gx_v, atom_v, *rest):
    bufs = rest[:NBUF]
    sems = rest[NBUF:2 * NBUF]
    cid = lax.axis_index("c")
    sid = lax.axis_index("s")
    wid = sid * NC + cid
    base = wid * CHUNK

    pltpu.sync_copy(atom_hbm.at[wid], atom_v)
    pltpu.sync_copy(w_hbm.at[pl.ds(base, CHUNK)], w_v)
    pltpu.sync_copy(g_hbm.at[pl.ds(base, CHUNK)], gx_v)

    def fire(b, i):
        pltpu.async_copy(node_hbm.at[atom_v.at[b]], bufs[i], sems[i])

    def drain(i):
        pltpu.make_async_copy(node_hbm.at[atom_v.at[0]], bufs[i],
                              sems[i]).wait()

    for i in range(NBUF):
        fire(i, i)

    zv = jnp.zeros((L,), jnp.float32)

    def z_body(i, _):
        acc_v[pl.ds(i * L, L)] = zv
        return 0

    lax.fori_loop(0, (G * D) // L, z_body, 0)

    def process(b, buf):
        def row(r, _):
            e = b * B + r
            g16 = plsc.load_gather(gx_v, [jnp.broadcast_to(e, (L,))])
            w16 = plsc.load_gather(w_v, [jnp.broadcast_to(e, (L,))])
            idx0 = g16 + _iota()
            for q in range(D // 32):
                v = buf[r, pl.ds(q * L, L)]
                lo = plsc.bitcast(v << 16, jnp.float32)
                hi = plsc.bitcast(v & jnp.int32(-65536), jnp.float32)
                col0 = q * L
                plsc.addupdate_scatter(acc_v, [idx0 + col0], lo * w16)
                plsc.addupdate_scatter(acc_v, [idx0 + (col0 + 128)], hi * w16)
            return 0

        lax.fori_loop(0, B, row, 0)

    def ring(t, _):
        for i in range(NBUF):
            b = t * NBUF + i
            drain(i)
            process(b, bufs[i])
            fire(b + NBUF, i)
        return 0

    lax.fori_loop(0, NB // NBUF, ring, 0)
    for i in range(NBUF):
        drain(i)

    pltpu.sync_copy(acc_v, part_hbm.at[wid])


@functools.lru_cache(maxsize=None)
def _build_prep():
    return functools.partial(
        pl.kernel,
        out_type=[
            jax.ShapeDtypeStruct((M_PAD,), jnp.float32),
            jax.ShapeDtypeStruct((M_PAD,), jnp.int32),
            jax.ShapeDtypeStruct((NC, G), jnp.float32),
        ],
        mesh=plsc.VectorSubcoreMesh(core_axis_name="c", subcore_axis_name="s",
                                    num_cores=NC, num_subcores=NS),
        compiler_params=pltpu.CompilerParams(needs_layout_passes=False),
        scratch_types=[
            pltpu.VMEM((M_PAD,), jnp.int32),
            pltpu.VMEM((F,), jnp.int32),
            pltpu.VMEM((FR,), jnp.float32),
            pltpu.VMEM((F_PAD,), jnp.float32),
            pltpu.VMEM((CHUNK,), jnp.float32),
            pltpu.VMEM((CHUNK,), jnp.int32),
            pltpu.VMEM((L,), jnp.float32),
            pltpu.VMEM_SHARED((F_PAD,), jnp.float32),
        ],
    )(_prep_body)


@functools.lru_cache(maxsize=None)
def _build_main():
    return functools.partial(
        pl.kernel,
        out_type=jax.ShapeDtypeStruct((NW, G * D), jnp.float32),
        mesh=plsc.VectorSubcoreMesh(core_axis_name="c", subcore_axis_name="s",
                                    num_cores=NC, num_subcores=NS),
        compiler_params=pltpu.CompilerParams(needs_layout_passes=False),
        scratch_types=(
            [
                pltpu.VMEM((G * D,), jnp.float32),
                pltpu.VMEM((CHUNK,), jnp.float32),
                pltpu.VMEM((CHUNK,), jnp.int32),
                pltpu.VMEM((NB + NBUF, B), jnp.int32),
            ]
            + [pltpu.VMEM((B, D // 2), jnp.int32) for _ in range(NBUF)]
            + [pltpu.SemaphoreType.DMA for _ in range(NBUF)]
        ),
    )(_main_body)


def _cast_body(x_ref, o_ref):
    lobits = lax.bitcast_convert_type(
        x_ref[:, :128].astype(jnp.bfloat16).astype(jnp.float32), jnp.uint32)
    hibits = lax.bitcast_convert_type(
        x_ref[:, 128:].astype(jnp.bfloat16).astype(jnp.float32), jnp.uint32)
    o_ref[...] = ((lobits >> 16) | (hibits & jnp.uint32(0xFFFF0000))
                  ).astype(jnp.int32)


_CAST_ROWS = 2000
_cast = pl.pallas_call(
    _cast_body,
    grid=(N_NODES // _CAST_ROWS,),
    in_specs=[pl.BlockSpec((_CAST_ROWS, D), lambda i: (i, 0))],
    out_specs=pl.BlockSpec((_CAST_ROWS, D // 2), lambda i: (i, 0)),
    out_shape=jax.ShapeDtypeStruct((N_NODES, D // 2), jnp.int32),
)


def _combine_body(part_ref, gcnt_ref, out_ref):
    cnt = jnp.maximum(gcnt_ref[0], 1.0)
    out_ref[...] = jnp.sum(part_ref[...], axis=0) / cnt[:, None]


_combine = pl.pallas_call(
    _combine_body,
    out_shape=jax.ShapeDtypeStruct((G, D), jnp.float32),
)


def kernel(node_feature, group_atom_idxs_1d, macro_node_scatter_idxs,
           fragment_batch_ids):
    seg = jnp.pad(macro_node_scatter_idxs.astype(jnp.int32),
                  (0, M_PAD - M), constant_values=F)
    atom = jnp.pad(group_atom_idxs_1d.astype(jnp.int32),
                   (0, M_PAD - M)).reshape(NW, NB, B)
    atom = jnp.concatenate(
        [atom, jnp.zeros((NW, NBUF, B), jnp.int32)], axis=1)
    bids = fragment_batch_ids.astype(jnp.int32)

    w, gx, gcnt = _build_prep()(seg, bids)
    node_i32 = _cast(node_feature)
    part = _build_main()(node_i32, atom, w, gx)
    return _combine(part.reshape(NW, G, D), gcnt)
